# Initial kernel scaffold; baseline (speedup 1.0000x reference)
import jax
import jax.numpy as jnp
from jax.experimental import pallas as pl


def kernel(mem, idx, val):
    raise NotImplementedError("write your pallas kernel here")


# speedup vs baseline: 25.3464x; 25.3464x over previous
import functools

import jax
import jax.numpy as jnp
from jax import lax
from jax.experimental import pallas as pl
from jax.experimental.pallas import tpu as pltpu
from jax.experimental.pallas import tpu_sc as plsc

M = 1000000
D = 16
B = 16384
NC = 2
NS = 16
NW = NC * NS
LANES = 16
RANGE = 31256
TPAD = NW * RANGE
BPW = B // NW
NVREG = B // LANES


def _winner_body(idx_hbm, t_hbm, idx_v, tbl_v):
    wid = lax.axis_index("s") * NC + lax.axis_index("c")
    lo = wid * RANGE
    pltpu.sync_copy(idx_hbm, idx_v)

    def step(v, _):
        ids = idx_v[pl.ds(v * LANES, LANES)]
        j = v * LANES + lax.iota(jnp.int32, LANES)
        mask = (ids >= lo) & (ids < lo + RANGE)
        loc = jnp.where(mask, ids - lo, 0)
        plsc.store_scatter(tbl_v, [loc], j, mask=mask)
        r = plsc.load_gather(tbl_v, [loc], mask=mask)

        def cond(r):
            return jnp.any(mask & (r < j))

        def body(r):
            plsc.store_scatter(tbl_v, [loc], j, mask=mask & (r < j))
            return plsc.load_gather(tbl_v, [loc], mask=mask)

        lax.while_loop(cond, body, r)
        return _

    lax.fori_loop(0, NVREG, step, None)
    pltpu.sync_copy(tbl_v, t_hbm.at[pl.ds(lo, RANGE)])


def _readback_body(idx_hbm, val_hbm, t_hbm, out_hbm, idxs_v, t_v, rows_v, sem):
    wid = lax.axis_index("s") * NC + lax.axis_index("c")
    base = wid * BPW
    pltpu.sync_copy(idx_hbm.at[pl.ds(base, BPW)], idxs_v)
    pltpu.async_copy(t_hbm.at[idxs_v], t_v, sem).wait()
    pltpu.async_copy(val_hbm.at[t_v], rows_v, sem).wait()
    pltpu.sync_copy(rows_v, out_hbm.at[pl.ds(base, BPW)])


def kernel(mem, idx, val):
    del mem
    mesh = plsc.VectorSubcoreMesh(core_axis_name="c", subcore_axis_name="s")

    winner = pl.kernel(
        _winner_body,
        out_type=jax.ShapeDtypeStruct((TPAD,), jnp.int32),
        mesh=mesh,
        compiler_params=pltpu.CompilerParams(needs_layout_passes=False),
        scratch_types=[
            pltpu.VMEM((B,), jnp.int32),
            pltpu.VMEM((RANGE,), jnp.int32),
        ],
    )
    t = winner(idx)

    readback = pl.kernel(
        _readback_body,
        out_type=jax.ShapeDtypeStruct((B, D), jnp.float32),
        mesh=mesh,
        compiler_params=pltpu.CompilerParams(use_tc_tiling_on_sc=False),
        scratch_types=[
            pltpu.VMEM((BPW,), jnp.int32),
            pltpu.VMEM((BPW,), jnp.int32),
            pltpu.VMEM((BPW, D), jnp.float32),
            pltpu.SemaphoreType.DMA,
        ],
    )
    return readback(idx, val, t)


# speedup vs baseline: 32.3629x; 1.2768x over previous
import functools

import jax
import jax.numpy as jnp
from jax import lax
from jax.experimental import pallas as pl
from jax.experimental.pallas import tpu as pltpu
from jax.experimental.pallas import tpu_sc as plsc

M = 1000000
D = 16
B = 16384
NC = 2
NS = 16
NW = NC * NS
LANES = 16
RANGE = 31256
TPAD = NW * RANGE
BPW = B // NW
NVREG = B // LANES


def _winner_body(idx_hbm, t_hbm, idx_v, tbl_v):
    wid = lax.axis_index("s") * NC + lax.axis_index("c")
    lo = wid * RANGE
    pltpu.sync_copy(idx_hbm, idx_v)

    def step(v, _):
        ids = idx_v[pl.ds(v * LANES, LANES)]
        j = v * LANES + lax.iota(jnp.int32, LANES)
        mask = (ids >= lo) & (ids < lo + RANGE)
        unused_cnt, last = plsc.scan_count(ids, mask=mask)
        keep = mask & last
        loc = jnp.where(keep, ids - lo, 0)
        plsc.store_scatter(tbl_v, [loc], j, mask=keep)
        return _

    lax.fori_loop(0, NVREG, step, None)
    pltpu.sync_copy(tbl_v, t_hbm.at[pl.ds(lo, RANGE)])


def _readback_body(idx_hbm, val_hbm, t_hbm, out_hbm, idxs_v, t_v, rows_v, sem):
    wid = lax.axis_index("s") * NC + lax.axis_index("c")
    base = wid * BPW
    pltpu.sync_copy(idx_hbm.at[pl.ds(base, BPW)], idxs_v)
    pltpu.async_copy(t_hbm.at[idxs_v], t_v, sem).wait()
    pltpu.async_copy(val_hbm.at[t_v], rows_v, sem).wait()
    pltpu.sync_copy(rows_v, out_hbm.at[pl.ds(base, BPW)])


def kernel(mem, idx, val):
    del mem
    mesh = plsc.VectorSubcoreMesh(core_axis_name="c", subcore_axis_name="s")

    winner = pl.kernel(
        _winner_body,
        out_type=jax.ShapeDtypeStruct((TPAD,), jnp.int32),
        mesh=mesh,
        compiler_params=pltpu.CompilerParams(needs_layout_passes=False),
        scratch_types=[
            pltpu.VMEM((B,), jnp.int32),
            pltpu.VMEM((RANGE,), jnp.int32),
        ],
    )
    t = winner(idx)

    readback = pl.kernel(
        _readback_body,
        out_type=jax.ShapeDtypeStruct((B, D), jnp.float32),
        mesh=mesh,
        compiler_params=pltpu.CompilerParams(use_tc_tiling_on_sc=False),
        scratch_types=[
            pltpu.VMEM((BPW,), jnp.int32),
            pltpu.VMEM((BPW,), jnp.int32),
            pltpu.VMEM((BPW, D), jnp.float32),
            pltpu.SemaphoreType.DMA,
        ],
    )
    return readback(idx, val, t)


# speedup vs baseline: 32.3630x; 1.0000x over previous
import functools

import jax
import jax.numpy as jnp
from jax import lax
from jax.experimental import pallas as pl
from jax.experimental.pallas import tpu as pltpu
from jax.experimental.pallas import tpu_sc as plsc

M = 1000000
D = 16
B = 16384
NC = 2
NS = 16
NW = NC * NS
LANES = 16
RANGE = 31256
TPAD = NW * RANGE
BPW = B // NW
NVREG = B // LANES


def _winner_body(idx_hbm, t_hbm, idx_v, tbl_v):
    wid = lax.axis_index("s") * NC + lax.axis_index("c")
    lo = wid * RANGE
    pltpu.sync_copy(idx_hbm, idx_v)

    def step(g, _):
        for k in range(4):
            v = g * 4 + k
            ids = idx_v[pl.ds(v * LANES, LANES)]
            j = v * LANES + lax.iota(jnp.int32, LANES)
            mask = (ids >= lo) & (ids < lo + RANGE)
            unused_cnt, last = plsc.scan_count(ids, mask=mask)
            keep = mask & last
            loc = jnp.where(keep, ids - lo, 0)
            plsc.store_scatter(tbl_v, [loc], j, mask=keep)
        return _

    lax.fori_loop(0, NVREG // 4, step, None)
    pltpu.sync_copy(tbl_v, t_hbm.at[pl.ds(lo, RANGE)])


def _readback_body(idx_hbm, val_hbm, t_hbm, out_hbm, idxs_v, t_v, rows_v, sem):
    wid = lax.axis_index("s") * NC + lax.axis_index("c")
    base = wid * BPW
    pltpu.sync_copy(idx_hbm.at[pl.ds(base, BPW)], idxs_v)
    pltpu.async_copy(t_hbm.at[idxs_v], t_v, sem).wait()
    pltpu.async_copy(val_hbm.at[t_v], rows_v, sem).wait()
    pltpu.sync_copy(rows_v, out_hbm.at[pl.ds(base, BPW)])


def kernel(mem, idx, val):
    del mem
    mesh = plsc.VectorSubcoreMesh(core_axis_name="c", subcore_axis_name="s")

    winner = pl.kernel(
        _winner_body,
        out_type=jax.ShapeDtypeStruct((TPAD,), jnp.int32),
        mesh=mesh,
        compiler_params=pltpu.CompilerParams(needs_layout_passes=False),
        scratch_types=[
            pltpu.VMEM((B,), jnp.int32),
            pltpu.VMEM((RANGE,), jnp.int32),
        ],
    )
    t = winner(idx)

    readback = pl.kernel(
        _readback_body,
        out_type=jax.ShapeDtypeStruct((B, D), jnp.float32),
        mesh=mesh,
        compiler_params=pltpu.CompilerParams(use_tc_tiling_on_sc=False),
        scratch_types=[
            pltpu.VMEM((BPW,), jnp.int32),
            pltpu.VMEM((BPW,), jnp.int32),
            pltpu.VMEM((BPW, D), jnp.float32),
            pltpu.SemaphoreType.DMA,
        ],
    )
    return readback(idx, val, t)
